# Initial kernel scaffold; baseline (speedup 1.0000x reference)
import functools

import jax
import jax.numpy as jnp
from jax import lax
from jax.experimental import pallas as pl
from jax.experimental.pallas import tpu as pltpu
from jax.experimental.pallas import tpu_sc as plsc

N = 10000
E = 320000
D = 128
NCLS = 40

NC = 2
NS = 16
NW = NC * NS
L = 16

NPAD = 10240
DUMP = N
EB = 128
EPW_H = E // NW
E_PAD = ((E + NW * EB - 1) // (NW * EB)) * NW * EB
EPW = E_PAD // NW
BPW = EPW // EB
RPT = NPAD // NS

BR = 256
GRID = NPAD // BR

_MESH = plsc.VectorSubcoreMesh(core_axis_name="c", subcore_axis_name="s")


def _hist_body(dst_hbm, zcol_hbm, out_hbm, dst_v, hist_v):
    c = lax.axis_index("c")
    s = lax.axis_index("s")
    wid = s * NC + c
    pltpu.sync_copy(zcol_hbm, hist_v)
    pltpu.sync_copy(dst_hbm.at[pl.ds(wid * EPW_H, EPW_H)], dst_v)
    ones = jnp.full((L,), 1.0, jnp.float32)
    zidx = jnp.zeros((L,), jnp.int32)

    @pl.loop(0, EPW_H // L)
    def _(i):
        d = dst_v[pl.ds(i * L, L)]
        plsc.addupdate_scatter(hist_v, [d, zidx], ones)

    pltpu.sync_copy(hist_v, out_hbm.at[wid])


def _hist_call(dst):
    zcol = jnp.zeros((NPAD, 1), jnp.float32)
    return pl.kernel(
        _hist_body,
        out_type=jax.ShapeDtypeStruct((NW, NPAD, 1), jnp.float32),
        mesh=_MESH,
        scratch_types=[
            pltpu.VMEM((EPW_H,), jnp.int32),
            pltpu.VMEM((NPAD, 1), jnp.float32),
        ],
    )(dst, zcol)


def _pass_body(table_hbm, src_hbm, dst_hbm, zrows_hbm, out_hbm,
               sidx_v, didx_v, rows_v, acc, sem):
    c = lax.axis_index("c")
    s = lax.axis_index("s")
    wid = s * NC + c
    pltpu.sync_copy(zrows_hbm.at[pl.ds(s * RPT, RPT)], acc.at[pl.ds(s * RPT, RPT)])
    plsc.subcore_barrier()

    base0 = wid * EPW

    @pl.loop(0, BPW)
    def _(b):
        off = base0 + b * EB
        pltpu.sync_copy(src_hbm.at[pl.ds(off, EB)], sidx_v)
        pltpu.sync_copy(dst_hbm.at[pl.ds(off, EB)], didx_v)
        pltpu.async_copy(table_hbm.at[sidx_v], rows_v, sem).wait()
        pltpu.sync_copy(rows_v, acc.at[didx_v], add=True)

    plsc.subcore_barrier()
    pltpu.sync_copy(acc.at[pl.ds(s * RPT, RPT)],
                    out_hbm.at[c, pl.ds(s * RPT, RPT)])


def _pass_call(table, src_p, dst_p, zrows):
    return pl.kernel(
        _pass_body,
        out_type=jax.ShapeDtypeStruct((NC, NPAD, D), jnp.float32),
        mesh=_MESH,
        scratch_types=[
            pltpu.VMEM((EB,), jnp.int32),
            pltpu.VMEM((EB,), jnp.int32),
            pltpu.VMEM((EB, D), jnp.float32),
            pltpu.VMEM_SHARED((NPAD, D), jnp.float32),
            pltpu.SemaphoreType.DMA,
        ],
    )(table, src_p, dst_p, zrows)


def _prep_body(parts_ref, x_ref, dinv_ref, xs_ref):
    deg = 1.0 + jnp.sum(parts_ref[...], axis=0)
    dinv = lax.rsqrt(deg)
    dinv_ref[...] = dinv
    xs_ref[...] = x_ref[...] * dinv


def _prep_call(hist_parts, x_pad):
    return pl.pallas_call(
        _prep_body,
        grid=(GRID,),
        in_specs=[
            pl.BlockSpec((NW, BR, 1), lambda i: (0, i, 0)),
            pl.BlockSpec((BR, D), lambda i: (i, 0)),
        ],
        out_specs=[
            pl.BlockSpec((BR, 1), lambda i: (i, 0)),
            pl.BlockSpec((BR, D), lambda i: (i, 0)),
        ],
        out_shape=[
            jax.ShapeDtypeStruct((NPAD, 1), jnp.float32),
            jax.ShapeDtypeStruct((NPAD, D), jnp.float32),
        ],
    )(hist_parts, x_pad)


def _combine1_body(parts_ref, x_ref, dinv_ref, w_ref, b_ref, z_ref, zs_ref):
    dinv = dinv_ref[...]
    p = parts_ref[0] + parts_ref[1]
    pre = dinv * p + (dinv * dinv) * x_ref[...]
    g = jnp.dot(pre, w_ref[...], preferred_element_type=jnp.float32) + b_ref[...]
    z = jnp.maximum(g, 0.0)
    z_ref[...] = z
    zs_ref[...] = dinv * z


def _combine1_call(parts, x_pad, dinv, W1, b1):
    return pl.pallas_call(
        _combine1_body,
        grid=(GRID,),
        in_specs=[
            pl.BlockSpec((NC, BR, D), lambda i: (0, i, 0)),
            pl.BlockSpec((BR, D), lambda i: (i, 0)),
            pl.BlockSpec((BR, 1), lambda i: (i, 0)),
            pl.BlockSpec((D, D), lambda i: (0, 0)),
            pl.BlockSpec((1, D), lambda i: (0, 0)),
        ],
        out_specs=[
            pl.BlockSpec((BR, D), lambda i: (i, 0)),
            pl.BlockSpec((BR, D), lambda i: (i, 0)),
        ],
        out_shape=[
            jax.ShapeDtypeStruct((NPAD, D), jnp.float32),
            jax.ShapeDtypeStruct((NPAD, D), jnp.float32),
        ],
    )(parts, x_pad, dinv, W1, b1.reshape(1, D))


def _combine2_body(parts_ref, z1_ref, dinv_ref, w2_ref, b2_ref,
                   w3_ref, b3_ref, w4_ref, b4_ref, out_ref):
    dinv = dinv_ref[...]
    p = parts_ref[0] + parts_ref[1]
    pre = dinv * p + (dinv * dinv) * z1_ref[...]
    g2 = jnp.dot(pre, w2_ref[...], preferred_element_type=jnp.float32) + b2_ref[...]
    h = jnp.maximum(
        jnp.dot(g2, w3_ref[...], preferred_element_type=jnp.float32) + b3_ref[...],
        0.0)
    out_ref[...] = (jnp.dot(h, w4_ref[...], preferred_element_type=jnp.float32)
                    + b4_ref[...])


def _combine2_call(parts, z1, dinv, W2, b2, W3, b3, W4, b4):
    return pl.pallas_call(
        _combine2_body,
        grid=(GRID,),
        in_specs=[
            pl.BlockSpec((NC, BR, D), lambda i: (0, i, 0)),
            pl.BlockSpec((BR, D), lambda i: (i, 0)),
            pl.BlockSpec((BR, 1), lambda i: (i, 0)),
            pl.BlockSpec((D, D), lambda i: (0, 0)),
            pl.BlockSpec((1, D), lambda i: (0, 0)),
            pl.BlockSpec((D, D), lambda i: (0, 0)),
            pl.BlockSpec((1, D), lambda i: (0, 0)),
            pl.BlockSpec((D, NCLS), lambda i: (0, 0)),
            pl.BlockSpec((1, NCLS), lambda i: (0, 0)),
        ],
        out_specs=[pl.BlockSpec((BR, NCLS), lambda i: (i, 0))],
        out_shape=[jax.ShapeDtypeStruct((NPAD, NCLS), jnp.float32)],
    )(parts, z1, dinv, W2, b2.reshape(1, D), W3, b3.reshape(1, D),
      W4, b4.reshape(1, NCLS))


def kernel(x, edge_index, W1, b1, W2, b2, W3, b3, W4, b4):
    src = edge_index[0]
    dst = edge_index[1]
    pad = jnp.full((E_PAD - E,), DUMP, jnp.int32)
    src_p = jnp.concatenate([src, pad])
    dst_p = jnp.concatenate([dst, pad])
    x_pad = jnp.pad(x, ((0, NPAD - N), (0, 0)))
    zrows = jnp.zeros((NPAD, D), jnp.float32)

    hist_parts = _hist_call(dst)
    dinv, xs1 = _prep_call(hist_parts, x_pad)

    parts1 = _pass_call(xs1, src_p, dst_p, zrows)
    z1, z1s = _combine1_call(parts1, x_pad, dinv, W1, b1)

    parts2 = _pass_call(z1s, src_p, dst_p, zrows)
    out_pad, = _combine2_call(parts2, z1, dinv, W2, b2, W3, b3, W4, b4)
    return out_pad[:N]


# speedup vs baseline: 9.4899x; 9.4899x over previous
import functools

import jax
import jax.numpy as jnp
from jax import lax
from jax.experimental import pallas as pl
from jax.experimental.pallas import tpu as pltpu
from jax.experimental.pallas import tpu_sc as plsc

N = 10000
E = 320000
D = 128
NCLS = 40

NC = 2
NS = 16
NW = NC * NS
L = 16

NPAD = 10240
DUMP = N
EB = 128
EPW_H = E // NW
E_PAD = ((E + NW * EB - 1) // (NW * EB)) * NW * EB
EPW = E_PAD // NW
BPW = EPW // EB
RPT = NPAD // NS

BR = 256
GRID = NPAD // BR

_MESH = plsc.VectorSubcoreMesh(core_axis_name="c", subcore_axis_name="s")


def _hist_body(dst_hbm, zcol_hbm, out_hbm, dst_v, hist_v):
    c = lax.axis_index("c")
    s = lax.axis_index("s")
    wid = s * NC + c
    pltpu.sync_copy(zcol_hbm, hist_v)
    pltpu.sync_copy(dst_hbm.at[pl.ds(wid * EPW_H, EPW_H)], dst_v)
    ones = jnp.full((L,), 1.0, jnp.float32)

    @pl.loop(0, EPW_H // L)
    def _(i):
        d = dst_v[pl.ds(i * L, L)]
        plsc.addupdate_scatter(hist_v, [d], ones)

    pltpu.sync_copy(hist_v, out_hbm.at[wid])


def _hist_call(dst):
    zcol = jnp.zeros((NPAD,), jnp.float32)
    return pl.kernel(
        _hist_body,
        out_type=jax.ShapeDtypeStruct((NW, NPAD), jnp.float32),
        mesh=_MESH,
        scratch_types=[
            pltpu.VMEM((EPW_H,), jnp.int32),
            pltpu.VMEM((NPAD,), jnp.float32),
        ],
        compiler_params=pltpu.CompilerParams(needs_layout_passes=False),
    )(dst, zcol)


def _pass_body(table_hbm, src_hbm, dst_hbm, zrows_hbm, out_hbm,
               sidx_v, didx_v, rows_v, acc, sem):
    c = lax.axis_index("c")
    s = lax.axis_index("s")
    wid = s * NC + c
    pltpu.sync_copy(zrows_hbm.at[pl.ds(s * RPT, RPT)], acc.at[pl.ds(s * RPT, RPT)])
    plsc.subcore_barrier()

    base0 = wid * EPW

    @pl.loop(0, BPW)
    def _(b):
        off = base0 + b * EB
        pltpu.sync_copy(src_hbm.at[pl.ds(off, EB)], sidx_v)
        pltpu.sync_copy(dst_hbm.at[pl.ds(off, EB)], didx_v)
        pltpu.async_copy(table_hbm.at[sidx_v], rows_v, sem).wait()
        pltpu.sync_copy(rows_v, acc.at[didx_v], add=True)

    plsc.subcore_barrier()
    pltpu.sync_copy(acc.at[pl.ds(s * RPT, RPT)],
                    out_hbm.at[c, pl.ds(s * RPT, RPT)])


def _pass_call(table, src_p, dst_p, zrows):
    return pl.kernel(
        _pass_body,
        out_type=jax.ShapeDtypeStruct((NC, NPAD, D), jnp.float32),
        mesh=_MESH,
        scratch_types=[
            pltpu.VMEM((EB,), jnp.int32),
            pltpu.VMEM((EB,), jnp.int32),
            pltpu.VMEM((EB, D), jnp.float32),
            pltpu.VMEM_SHARED((NPAD, D), jnp.float32),
            pltpu.SemaphoreType.DMA,
        ],
    )(table, src_p, dst_p, zrows)


def _prep_body(parts_ref, x_ref, dinv_ref, xs_ref):
    deg = 1.0 + jnp.sum(parts_ref[...], axis=0)
    dinv = lax.rsqrt(deg)
    dinv_ref[...] = dinv
    xs_ref[...] = x_ref[...] * dinv


def _prep_call(hist_parts, x_pad):
    return pl.pallas_call(
        _prep_body,
        grid=(GRID,),
        in_specs=[
            pl.BlockSpec((NW, BR, 1), lambda i: (0, i, 0)),
            pl.BlockSpec((BR, D), lambda i: (i, 0)),
        ],
        out_specs=[
            pl.BlockSpec((BR, 1), lambda i: (i, 0)),
            pl.BlockSpec((BR, D), lambda i: (i, 0)),
        ],
        out_shape=[
            jax.ShapeDtypeStruct((NPAD, 1), jnp.float32),
            jax.ShapeDtypeStruct((NPAD, D), jnp.float32),
        ],
    )(hist_parts, x_pad)


def _combine1_body(parts_ref, x_ref, dinv_ref, w_ref, b_ref, z_ref, zs_ref):
    dinv = dinv_ref[...]
    p = parts_ref[0] + parts_ref[1]
    pre = dinv * p + (dinv * dinv) * x_ref[...]
    g = jnp.dot(pre, w_ref[...], preferred_element_type=jnp.float32) + b_ref[...]
    z = jnp.maximum(g, 0.0)
    z_ref[...] = z
    zs_ref[...] = dinv * z


def _combine1_call(parts, x_pad, dinv, W1, b1):
    return pl.pallas_call(
        _combine1_body,
        grid=(GRID,),
        in_specs=[
            pl.BlockSpec((NC, BR, D), lambda i: (0, i, 0)),
            pl.BlockSpec((BR, D), lambda i: (i, 0)),
            pl.BlockSpec((BR, 1), lambda i: (i, 0)),
            pl.BlockSpec((D, D), lambda i: (0, 0)),
            pl.BlockSpec((1, D), lambda i: (0, 0)),
        ],
        out_specs=[
            pl.BlockSpec((BR, D), lambda i: (i, 0)),
            pl.BlockSpec((BR, D), lambda i: (i, 0)),
        ],
        out_shape=[
            jax.ShapeDtypeStruct((NPAD, D), jnp.float32),
            jax.ShapeDtypeStruct((NPAD, D), jnp.float32),
        ],
    )(parts, x_pad, dinv, W1, b1.reshape(1, D))


def _combine2_body(parts_ref, z1_ref, dinv_ref, w2_ref, b2_ref,
                   w3_ref, b3_ref, w4_ref, b4_ref, out_ref):
    dinv = dinv_ref[...]
    p = parts_ref[0] + parts_ref[1]
    pre = dinv * p + (dinv * dinv) * z1_ref[...]
    g2 = jnp.dot(pre, w2_ref[...], preferred_element_type=jnp.float32) + b2_ref[...]
    h = jnp.maximum(
        jnp.dot(g2, w3_ref[...], preferred_element_type=jnp.float32) + b3_ref[...],
        0.0)
    out_ref[...] = (jnp.dot(h, w4_ref[...], preferred_element_type=jnp.float32)
                    + b4_ref[...])


def _combine2_call(parts, z1, dinv, W2, b2, W3, b3, W4, b4):
    return pl.pallas_call(
        _combine2_body,
        grid=(GRID,),
        in_specs=[
            pl.BlockSpec((NC, BR, D), lambda i: (0, i, 0)),
            pl.BlockSpec((BR, D), lambda i: (i, 0)),
            pl.BlockSpec((BR, 1), lambda i: (i, 0)),
            pl.BlockSpec((D, D), lambda i: (0, 0)),
            pl.BlockSpec((1, D), lambda i: (0, 0)),
            pl.BlockSpec((D, D), lambda i: (0, 0)),
            pl.BlockSpec((1, D), lambda i: (0, 0)),
            pl.BlockSpec((D, NCLS), lambda i: (0, 0)),
            pl.BlockSpec((1, NCLS), lambda i: (0, 0)),
        ],
        out_specs=[pl.BlockSpec((BR, NCLS), lambda i: (i, 0))],
        out_shape=[jax.ShapeDtypeStruct((NPAD, NCLS), jnp.float32)],
    )(parts, z1, dinv, W2, b2.reshape(1, D), W3, b3.reshape(1, D),
      W4, b4.reshape(1, NCLS))


def kernel(x, edge_index, W1, b1, W2, b2, W3, b3, W4, b4):
    src = edge_index[0]
    dst = edge_index[1]
    pad = jnp.full((E_PAD - E,), DUMP, jnp.int32)
    src_p = jnp.concatenate([src, pad])
    dst_p = jnp.concatenate([dst, pad])
    x_pad = jnp.pad(x, ((0, NPAD - N), (0, 0)))
    zrows = jnp.zeros((NPAD, D), jnp.float32)

    hist_parts = _hist_call(dst).reshape(NW, NPAD, 1)
    dinv, xs1 = _prep_call(hist_parts, x_pad)

    parts1 = _pass_call(xs1, src_p, dst_p, zrows)
    z1, z1s = _combine1_call(parts1, x_pad, dinv, W1, b1)

    parts2 = _pass_call(z1s, src_p, dst_p, zrows)
    out_pad, = _combine2_call(parts2, z1, dinv, W2, b2, W3, b3, W4, b4)
    return out_pad[:N]
